# baseline (speedup 1.0000x reference)
import functools

import jax
import jax.numpy as jnp
from jax import lax
from jax.experimental import pallas as pl
from jax.experimental.pallas import tpu as pltpu
from jax.experimental.pallas import tpu_sc as plsc

_B = 16384
_F = 26
_K = 16
_V = 1000000
_NW = 32
_BR = _B // _NW
_NLB = _BR // 128
_NBUF = 4
_TPAD = _BR + 1


def _em_pallas(idx_t, table):
    mesh = plsc.VectorSubcoreMesh(core_axis_name="c", subcore_axis_name="s")

    @functools.partial(
        pl.kernel,
        mesh=mesh,
        out_type=jax.ShapeDtypeStruct((_F, 2, _B // 128, 8, 128),
                                      jnp.float32),
        scratch_types=[
            pltpu.VMEM((_F, _BR), jnp.int32),
            pltpu.VMEM((_NBUF, _NLB, 128, _K), jnp.float32),
            pltpu.SemaphoreType.DMA((_NBUF,)),
        ],
        compiler_params=pltpu.CompilerParams(
            use_tc_tiling_on_sc=False, needs_layout_passes=False
        ),
    )
    def k(idxt_hbm, tab_hbm, out_hbm, idx_v, rows_v, gsem):
        wid = lax.axis_index("s") * 2 + lax.axis_index("c")
        c0 = wid * _BR
        pltpu.sync_copy(idxt_hbm.at[:, pl.ds(c0, _BR)], idx_v)

        def fire(f):
            slot = lax.rem(f, _NBUF)
            for lb in range(_NLB):
                pltpu.make_async_copy(
                    tab_hbm.at[idx_v.at[f, pl.ds(lb * 128, 128)]],
                    rows_v.at[slot, lb],
                    gsem.at[slot],
                ).start()

        def consume(f, carry):
            slot = lax.rem(f, _NBUF)
            for lb in range(_NLB):
                pltpu.make_async_copy(
                    tab_hbm.at[idx_v.at[f, pl.ds(lb * 128, 128)]],
                    rows_v.at[slot, lb],
                    gsem.at[slot],
                ).wait()

            for kk in range(_K):
                pltpu.sync_copy(
                    rows_v.at[slot, :, :, kk],
                    out_hbm.at[f, kk // 8, pl.ds(wid * _NLB, _NLB), kk % 8],
                )
            return carry

        for f in range(_NBUF):
            fire(jnp.int32(f))

        def steady(f, carry):
            carry = consume(f, carry)
            fire(f + _NBUF)
            return carry

        lax.fori_loop(0, _F - _NBUF, steady, 0)
        lax.fori_loop(_F - _NBUF, _F, consume, 0)

    return k(idx_t, table)


def kernel(idx, val, table):
    if idx.dtype != jnp.int32:
        idx = idx.astype(jnp.int32)
    idx_t = idx.T
    out5d = _em_pallas(idx_t, table)
    em_w = (out5d.transpose(2, 4, 0, 1, 3)
            .reshape(_B, _F, _K))
    val_e = val[..., None]
    return (em_w, val_e)


# speedup vs baseline: 28.5487x; 28.5487x over previous
import functools

import jax
import jax.numpy as jnp
from jax import lax
from jax.experimental import pallas as pl
from jax.experimental.pallas import tpu as pltpu
from jax.experimental.pallas import tpu_sc as plsc

_B = 16384
_F = 26
_K = 16
_V = 1000000
_NW = 32
_BR = _B // _NW
_NLB = _BR // 128
_NBUF = 4
_TPAD = _BR + 1


def _em_pallas(idx_t, table):
    mesh = plsc.VectorSubcoreMesh(core_axis_name="c", subcore_axis_name="s")

    @functools.partial(
        pl.kernel,
        mesh=mesh,
        out_type=jax.ShapeDtypeStruct((_F, 2, _B // 128, 8, 128),
                                      jnp.float32),
        scratch_types=[
            pltpu.VMEM((_F, _BR), jnp.int32),
            pltpu.VMEM((_NBUF, _BR, _K), jnp.float32),
            pltpu.VMEM((_K, _TPAD), jnp.float32),
            pltpu.SemaphoreType.DMA((_NBUF,)),
        ],
        compiler_params=pltpu.CompilerParams(
            use_tc_tiling_on_sc=False, needs_layout_passes=False
        ),
    )
    def k(idxt_hbm, tab_hbm, out_hbm, idx_v, rows_v, t_v, gsem):
        wid = lax.axis_index("s") * 2 + lax.axis_index("c")
        c0 = wid * _BR
        pltpu.sync_copy(idxt_hbm.at[:, pl.ds(c0, _BR)], idx_v)

        def fire(f):
            slot = lax.rem(f, _NBUF)
            pltpu.make_async_copy(
                tab_hbm.at[idx_v.at[f]], rows_v.at[slot], gsem.at[slot]
            ).start()

        def consume(f, carry):
            slot = lax.rem(f, _NBUF)
            pltpu.make_async_copy(
                tab_hbm.at[idx_v.at[f]], rows_v.at[slot], gsem.at[slot]
            ).wait()

            iota16 = lax.iota(jnp.int32, 16)
            for i in range(_BR):
                v = rows_v[slot, i, :]
                plsc.store_scatter(
                    t_v, [iota16, jnp.full((16,), i, jnp.int32)], v
                )

            for kb in range(2):
                for lb in range(_NLB):
                    pltpu.sync_copy(
                        t_v.at[pl.ds(kb * 8, 8), pl.ds(lb * 128, 128)],
                        out_hbm.at[f, kb, wid * _NLB + lb],
                    )
            return carry

        for f in range(_NBUF):
            fire(jnp.int32(f))

        def steady(f, carry):
            carry = consume(f, carry)
            fire(f + _NBUF)
            return carry

        lax.fori_loop(0, _F - _NBUF, steady, 0)
        lax.fori_loop(_F - _NBUF, _F, consume, 0)

    return k(idx_t, table)


def kernel(idx, val, table):
    if idx.dtype != jnp.int32:
        idx = idx.astype(jnp.int32)
    idx_t = idx.T
    out5d = _em_pallas(idx_t, table)
    em_w = (out5d.transpose(2, 4, 0, 1, 3)
            .reshape(_B, _F, _K))
    val_e = val[..., None]
    return (em_w, val_e)


# speedup vs baseline: 29.2284x; 1.0238x over previous
import functools

import jax
import jax.numpy as jnp
from jax import lax
from jax.experimental import pallas as pl
from jax.experimental.pallas import tpu as pltpu
from jax.experimental.pallas import tpu_sc as plsc

_B = 16384
_F = 26
_K = 16
_V = 1000000
_NW = 32
_BR = _B // _NW
_NLB = _BR // 128
_NBUF = 4
_TPAD = _BR + 1


def _em_pallas(idx_t, table):
    mesh = plsc.VectorSubcoreMesh(core_axis_name="c", subcore_axis_name="s")

    @functools.partial(
        pl.kernel,
        mesh=mesh,
        out_type=jax.ShapeDtypeStruct((_F, 2, _B // 128, 8, 128),
                                      jnp.float32),
        scratch_types=[
            pltpu.VMEM((_F, _BR), jnp.int32),
            pltpu.VMEM((_NBUF, _BR, _K), jnp.float32),
            pltpu.VMEM((_K, _TPAD), jnp.float32),
            pltpu.SemaphoreType.DMA((_NBUF,)),
            pltpu.SemaphoreType.DMA,
        ],
        compiler_params=pltpu.CompilerParams(
            use_tc_tiling_on_sc=False, needs_layout_passes=False
        ),
    )
    def k(idxt_hbm, tab_hbm, out_hbm, idx_v, rows_v, t_v, gsem, wsem):
        wid = lax.axis_index("s") * 2 + lax.axis_index("c")
        c0 = wid * _BR
        pltpu.sync_copy(idxt_hbm.at[:, pl.ds(c0, _BR)], idx_v)

        def fire(f):
            slot = lax.rem(f, _NBUF)
            pltpu.make_async_copy(
                tab_hbm.at[idx_v.at[f]], rows_v.at[slot], gsem.at[slot]
            ).start()

        def consume(f, carry):
            slot = lax.rem(f, _NBUF)
            pltpu.make_async_copy(
                tab_hbm.at[idx_v.at[f]], rows_v.at[slot], gsem.at[slot]
            ).wait()

            iota16 = lax.iota(jnp.int32, 16)
            for i in range(_BR):
                v = rows_v[slot, i, :]
                plsc.store_scatter(
                    t_v, [iota16, jnp.full((16,), i, jnp.int32)], v
                )

            for kb in range(2):
                for lb in range(_NLB):
                    pltpu.make_async_copy(
                        t_v.at[pl.ds(kb * 8, 8), pl.ds(lb * 128, 128)],
                        out_hbm.at[f, kb, wid * _NLB + lb],
                        wsem,
                    ).start()
            for kb in range(2):
                for lb in range(_NLB):
                    pltpu.make_async_copy(
                        t_v.at[pl.ds(kb * 8, 8), pl.ds(lb * 128, 128)],
                        out_hbm.at[f, kb, wid * _NLB + lb],
                        wsem,
                    ).wait()
            return carry

        for f in range(_NBUF):
            fire(jnp.int32(f))

        def steady(f, carry):
            carry = consume(f, carry)
            fire(f + _NBUF)
            return carry

        lax.fori_loop(0, _F - _NBUF, steady, 0)
        lax.fori_loop(_F - _NBUF, _F, consume, 0)

    return k(idx_t, table)


def kernel(idx, val, table):
    if idx.dtype != jnp.int32:
        idx = idx.astype(jnp.int32)
    idx_t = idx.T
    out5d = _em_pallas(idx_t, table)
    em_w = (out5d.transpose(2, 4, 0, 1, 3)
            .reshape(_B, _F, _K))
    val_e = val[..., None]
    return (em_w, val_e)
